# Initial kernel scaffold; baseline (speedup 1.0000x reference)
import jax
import jax.numpy as jnp
from jax.experimental import pallas as pl


def kernel(boxes, labels, scores):
    raise NotImplementedError("write your pallas kernel here")


# speedup vs baseline: 10.2393x; 10.2393x over previous
import jax
import jax.numpy as jnp
from jax.experimental import pallas as pl
from jax.experimental.pallas import tpu as pltpu

N = 5000
NP = 5120
B = 128
NB = NP // B
RB = 256
NR = NP // RB
IOU_T = 0.5
F32 = jnp.float32


def _sup_matrix(y1a, x1a, y2a, x2a, aa, la, y1b, x1b, y2b, x2b, ab, lb):
    iy1 = jnp.maximum(y1a, y1b)
    ix1 = jnp.maximum(x1a, x1b)
    iy2 = jnp.minimum(y2a, y2b)
    ix2 = jnp.minimum(x2a, x2b)
    ih = jnp.maximum(iy2 - iy1, 0.0)
    iw = jnp.maximum(ix2 - ix1, 0.0)
    inter = ih * iw
    union = aa + ab - inter
    iou = inter / jnp.maximum(union, 1e-9)
    return ((iou > IOU_T) & (la == lb)).astype(F32)


def _rank_permute_kernel(scol_f_ref, scol_b_ref, srow_f_ref, srow_b_ref,
                         box_ref, lab_ref,
                         rank_row_ref, rank_col_ref, sorted_ref):
    r = pl.program_id(0)

    scol = scol_f_ref[...]
    srow_b = srow_b_ref[...]
    jj = jax.lax.broadcasted_iota(jnp.int32, (NP, RB), 0)
    ii = jax.lax.broadcasted_iota(jnp.int32, (NP, RB), 1) + r * RB
    cmp_t = ((scol > srow_b) | ((scol == srow_b) & (jj < ii))).astype(F32)
    rank_row = jnp.sum(cmp_t, axis=0, keepdims=True)
    rank_row_ref[...] = rank_row

    scol_b = scol_b_ref[...]
    srow = srow_f_ref[...]
    jj2 = jax.lax.broadcasted_iota(jnp.int32, (RB, NP), 1)
    ii2 = jax.lax.broadcasted_iota(jnp.int32, (RB, NP), 0) + r * RB
    cmp_c = ((srow > scol_b) | ((srow == scol_b) & (jj2 < ii2))).astype(F32)
    rank_col_ref[...] = jnp.sum(cmp_c, axis=1, keepdims=True)

    b = box_ref[...]
    yc, xc = b[:, 0:1], b[:, 1:2]
    h, w = jnp.abs(b[:, 2:3]), jnp.abs(b[:, 3:4])
    y1 = yc - h / 2.0
    x1 = xc - w / 2.0
    y2 = yc + h / 2.0
    x2 = xc + w / 2.0
    area = h * w
    lab = lab_ref[...]
    packed = jnp.concatenate(
        [y1, x1, y2, x2, area, lab, jnp.zeros((RB, 2), F32)], axis=1)

    p_iota = jax.lax.broadcasted_iota(jnp.int32, (NP, RB), 0).astype(F32)
    pt = (p_iota == rank_row).astype(F32)
    contrib = jax.lax.dot_general(
        pt, packed, (((1,), (0,)), ((), ())),
        preferred_element_type=F32, precision=jax.lax.Precision.HIGHEST)

    @pl.when(r == 0)
    def _():
        sorted_ref[...] = jnp.zeros_like(sorted_ref)

    sorted_ref[...] += contrib


def _nms_kernel(col_ref, rowf_ref, rowb_ref, keep_ref, sbb_ref):
    b = pl.program_id(0)

    @pl.when(b == 0)
    def _():
        keep_ref[...] = jnp.ones_like(keep_ref)

    colb = col_ref[...]
    y1c, x1c, y2c, x2c = colb[:, 0:1], colb[:, 1:2], colb[:, 2:3], colb[:, 3:4]
    ac, lc = colb[:, 4:5], colb[:, 5:6]

    rowf = rowf_ref[...]
    s_full = _sup_matrix(y1c, x1c, y2c, x2c, ac, lc,
                         rowf[0:1, :], rowf[1:2, :], rowf[2:3, :],
                         rowf[3:4, :], rowf[4:5, :], rowf[5:6, :])

    rowb = rowb_ref[...]
    sbb_ref[...] = _sup_matrix(y1c, x1c, y2c, x2c, ac, lc,
                               rowb[0:1, :], rowb[1:2, :], rowb[2:3, :],
                               rowb[3:4, :], rowb[4:5, :], rowb[5:6, :])

    keep_old = keep_ref[...]
    sub_np = jax.lax.broadcasted_iota(jnp.int32, (NP, B), 0)
    lane_b2 = jax.lax.broadcasted_iota(jnp.int32, (NP, B), 1)
    ebt = (sub_np == lane_b2 + b * B).astype(F32)
    keep_blk = jax.lax.dot_general(
        keep_old, ebt, (((1,), (0,)), ((), ())),
        preferred_element_type=F32, precision=jax.lax.Precision.HIGHEST)

    lane_b = jax.lax.broadcasted_iota(jnp.int32, (1, B), 1)

    def body(i, kb):
        row = sbb_ref[pl.ds(i, 1), :]
        k_i = jnp.sum(kb * (lane_b == i).astype(F32))
        upd = row * (lane_b > i).astype(F32) * k_i
        return kb * (1.0 - upd)

    keep_blk = jax.lax.fori_loop(0, B, body, keep_blk)

    dead = jax.lax.dot_general(
        keep_blk, s_full, (((1,), (0,)), ((), ())),
        preferred_element_type=F32, precision=jax.lax.Precision.HIGHEST)
    dead = jnp.minimum(dead, 1.0)

    sub_b = jax.lax.broadcasted_iota(jnp.int32, (B, NP), 0)
    lane_np2 = jax.lax.broadcasted_iota(jnp.int32, (B, NP), 1)
    eb = (lane_np2 == sub_b + b * B).astype(F32)
    keep_blk_full = jax.lax.dot_general(
        keep_blk, eb, (((1,), (0,)), ((), ())),
        preferred_element_type=F32, precision=jax.lax.Precision.HIGHEST)

    lane_np = jax.lax.broadcasted_iota(jnp.int32, (1, NP), 1)
    future = (lane_np >= (b + 1) * B).astype(F32)
    inblk = ((lane_np >= b * B) & (lane_np < (b + 1) * B)).astype(F32)
    keep_new = keep_old * (1.0 - dead * future)
    keep_ref[...] = keep_new * (1.0 - inblk) + keep_blk_full


def _finalize_kernel(rank_col_ref, keep_col_ref, box_ref, sc_ref, out_ref):
    rank_col = rank_col_ref[...]
    lane_np = jax.lax.broadcasted_iota(jnp.int32, (RB, NP), 1).astype(F32)
    p_blk = (rank_col == lane_np).astype(F32)
    m_col = jax.lax.dot_general(
        p_blk, keep_col_ref[...], (((1,), (0,)), ((), ())),
        preferred_element_type=F32, precision=jax.lax.Precision.HIGHEST)
    b4 = box_ref[...] * m_col
    s1 = sc_ref[...] * m_col
    out_ref[...] = jnp.concatenate([b4, s1, jnp.zeros((RB, 3), F32)], axis=1)


def kernel(boxes, labels, scores):
    boxes = boxes.astype(F32)
    scores = scores.astype(F32)
    lab_f = labels.astype(F32)

    pad = NP - N
    boxes_p = jnp.concatenate([boxes, jnp.zeros((pad, 4), F32)], axis=0)
    scores_p = jnp.concatenate([scores, jnp.full((pad,), -1.0, F32)], axis=0)
    lab_p = jnp.concatenate([lab_f, jnp.full((pad,), -1.0, F32)], axis=0)

    scol = scores_p.reshape(NP, 1)
    srow = scores_p.reshape(1, NP)
    lcol = lab_p.reshape(NP, 1)

    rank_row, rank_col, sorted_col = pl.pallas_call(
        _rank_permute_kernel,
        grid=(NR,),
        in_specs=[
            pl.BlockSpec((NP, 1), lambda r: (0, 0)),
            pl.BlockSpec((RB, 1), lambda r: (r, 0)),
            pl.BlockSpec((1, NP), lambda r: (0, 0)),
            pl.BlockSpec((1, RB), lambda r: (0, r)),
            pl.BlockSpec((RB, 4), lambda r: (r, 0)),
            pl.BlockSpec((RB, 1), lambda r: (r, 0)),
        ],
        out_specs=[
            pl.BlockSpec((1, RB), lambda r: (0, r)),
            pl.BlockSpec((RB, 1), lambda r: (r, 0)),
            pl.BlockSpec((NP, 8), lambda r: (0, 0)),
        ],
        out_shape=[
            jax.ShapeDtypeStruct((1, NP), F32),
            jax.ShapeDtypeStruct((NP, 1), F32),
            jax.ShapeDtypeStruct((NP, 8), F32),
        ],
    )(scol, scol, srow, srow, boxes_p, lcol)

    sorted_row = sorted_col.T

    keep_row = pl.pallas_call(
        _nms_kernel,
        grid=(NB,),
        in_specs=[
            pl.BlockSpec((B, 8), lambda b: (b, 0)),
            pl.BlockSpec((8, NP), lambda b: (0, 0)),
            pl.BlockSpec((8, B), lambda b: (0, b)),
        ],
        out_specs=pl.BlockSpec((1, NP), lambda b: (0, 0)),
        out_shape=jax.ShapeDtypeStruct((1, NP), F32),
        scratch_shapes=[pltpu.VMEM((B, B), F32)],
    )(sorted_col, sorted_row, sorted_row)

    keep_col = keep_row.reshape(NP, 1)

    out8 = pl.pallas_call(
        _finalize_kernel,
        grid=(NR,),
        in_specs=[
            pl.BlockSpec((RB, 1), lambda r: (r, 0)),
            pl.BlockSpec((NP, 1), lambda r: (0, 0)),
            pl.BlockSpec((RB, 4), lambda r: (r, 0)),
            pl.BlockSpec((RB, 1), lambda r: (r, 0)),
        ],
        out_specs=pl.BlockSpec((RB, 8), lambda r: (r, 0)),
        out_shape=jax.ShapeDtypeStruct((NP, 8), F32),
    )(rank_col, keep_col, boxes_p, scol)

    return out8[:N, :5]


# speedup vs baseline: 55.9800x; 5.4672x over previous
import functools

import jax
import jax.numpy as jnp
from jax import lax
from jax.experimental import pallas as pl
from jax.experimental.pallas import tpu as pltpu
from jax.experimental.pallas import tpu_sc as plsc

N = 5000
NP = 5120
B = 256
NB = NP // B
RB = 512
NR = NP // RB
IOU_T = 0.5
F32 = jnp.float32

SC_NC, SC_NS, SC_L = 2, 16, 16
NW = SC_NC * SC_NS
CHUNK = NP // NW
SUB = 80
D = 128


def _sup_matrix(y1a, x1a, y2a, x2a, aa, la, y1b, x1b, y2b, x2b, ab, lb):
    iy1 = jnp.maximum(y1a, y1b)
    ix1 = jnp.maximum(x1a, x1b)
    iy2 = jnp.minimum(y2a, y2b)
    ix2 = jnp.minimum(x2a, x2b)
    ih = jnp.maximum(iy2 - iy1, 0.0)
    iw = jnp.maximum(ix2 - ix1, 0.0)
    inter = ih * iw
    union = aa + ab - inter
    iou = inter / jnp.maximum(union, 1e-9)
    return ((iou > IOU_T) & (la == lb)).astype(F32)


def _rank_permute_kernel(scol_f_ref, srow_b_ref, box_ref, lab_ref, sc_b_ref,
                         rank_row_ref, packed_ref):
    r = pl.program_id(0)

    scol = scol_f_ref[...]
    srow_b = srow_b_ref[...]
    jj = jax.lax.broadcasted_iota(jnp.int32, (NP, RB), 0)
    ii = jax.lax.broadcasted_iota(jnp.int32, (NP, RB), 1) + r * RB
    cmp_t = ((scol > srow_b) | ((scol == srow_b) & (jj < ii))).astype(jnp.bfloat16)
    ones_row = jnp.ones((1, NP), jnp.bfloat16)
    rank_row = jax.lax.dot_general(
        ones_row, cmp_t, (((1,), (0,)), ((), ())),
        preferred_element_type=F32)
    rank_row_ref[...] = rank_row

    b = box_ref[...]
    yc, xc = b[:, 0:1], b[:, 1:2]
    h, w = jnp.abs(b[:, 2:3]), jnp.abs(b[:, 3:4])
    y1 = yc - h / 2.0
    x1 = xc - w / 2.0
    y2 = yc + h / 2.0
    x2 = xc + w / 2.0
    area = h * w
    lab = lab_ref[...]
    sc_b = sc_b_ref[...]
    packed_ref[...] = jnp.concatenate(
        [y1, x1, y2, x2, area, lab, b, sc_b,
         jnp.zeros((RB, D - 11), F32)], axis=1)


def _nms_kernel(col_ref, rowf_ref, rowb_ref, keep_ref, keep_col_ref):
    b = pl.program_id(0)

    @pl.when(b == 0)
    def _():
        keep_ref[...] = jnp.ones_like(keep_ref)

    colb = col_ref[...]
    y1c, x1c, y2c, x2c = colb[:, 0:1], colb[:, 1:2], colb[:, 2:3], colb[:, 3:4]
    ac, lc = colb[:, 4:5], colb[:, 5:6]

    rowf = rowf_ref[...]
    s_full = _sup_matrix(y1c, x1c, y2c, x2c, ac, lc,
                         rowf[0:1, :], rowf[1:2, :], rowf[2:3, :],
                         rowf[3:4, :], rowf[4:5, :], rowf[5:6, :])

    rowb = rowb_ref[...]
    s_bb = _sup_matrix(y1c, x1c, y2c, x2c, ac, lc,
                       rowb[0:1, :], rowb[1:2, :], rowb[2:3, :],
                       rowb[3:4, :], rowb[4:5, :], rowb[5:6, :])
    sub_bb = jax.lax.broadcasted_iota(jnp.int32, (B, B), 0)
    lane_bb = jax.lax.broadcasted_iota(jnp.int32, (B, B), 1)
    s_ut = s_bb * (sub_bb < lane_bb).astype(F32)

    keep_old = keep_ref[...]
    keep_blk = keep_ref[0:1, pl.ds(b * B, B)]

    kb0 = keep_blk

    def cond(c):
        t, _, changed = c
        return changed & (t <= B)

    s_ut16 = s_ut.astype(jnp.bfloat16)

    def _step(kb):
        dead_b = jax.lax.dot_general(
            kb.astype(jnp.bfloat16), s_ut16, (((1,), (0,)), ((), ())),
            preferred_element_type=F32)
        return kb0 * (1.0 - jnp.minimum(dead_b, 1.0))

    def body(c):
        t, kb, _ = c
        kb_new = _step(_step(kb))
        return (t + 1, kb_new, jnp.any(kb_new != kb))

    _, keep_blk, _ = jax.lax.while_loop(
        cond, body, (jnp.int32(0), kb0, jnp.bool_(True)))

    dead = jax.lax.dot_general(
        keep_blk.astype(jnp.bfloat16), s_full.astype(jnp.bfloat16),
        (((1,), (0,)), ((), ())),
        preferred_element_type=F32)
    dead = jnp.minimum(dead, 1.0)

    lane_np = jax.lax.broadcasted_iota(jnp.int32, (1, NP), 1)
    future = (lane_np >= (b + 1) * B).astype(F32)
    keep_ref[...] = keep_old * (1.0 - dead * future)
    keep_ref[0:1, pl.ds(b * B, B)] = keep_blk
    keep_col_ref[...] = keep_blk.T


def _finalize_kernel(keep_col_ref, sorted_ref, out_ref):
    m_col = keep_col_ref[...]
    fields = sorted_ref[:, 6:11]
    out_ref[...] = jnp.concatenate(
        [fields * m_col, jnp.zeros((NP, D - 5), F32)], axis=1)


def _sc_row_scatter_call(table, idx3_i32):
    mesh = plsc.VectorSubcoreMesh(core_axis_name="c", subcore_axis_name="s")

    @functools.partial(
        pl.kernel, mesh=mesh,
        out_type=jax.ShapeDtypeStruct((NP, D), F32),
        scratch_types=[
            pltpu.VMEM((CHUNK // SUB, SUB), jnp.int32),
            pltpu.VMEM((CHUNK, D), F32),
            pltpu.SemaphoreType.DMA,
        ],
    )
    def k(table_hbm, idx_hbm, out_hbm, idx_v, rows_v, sem):
        wid = lax.axis_index("s") * SC_NC + lax.axis_index("c")
        base = wid * CHUNK
        pltpu.sync_copy(idx_hbm.at[wid], idx_v)
        pltpu.sync_copy(table_hbm.at[pl.ds(base, CHUNK)], rows_v)
        for c in range(CHUNK // SUB):
            pltpu.async_copy(rows_v.at[pl.ds(c * SUB, SUB)],
                             out_hbm.at[idx_v.at[c]], sem).wait()

    return k(table, idx3_i32)


def _sc_row_gather_call(table, idx_i32):
    mesh = plsc.VectorSubcoreMesh(core_axis_name="c", subcore_axis_name="s")

    @functools.partial(
        pl.kernel, mesh=mesh,
        out_type=jax.ShapeDtypeStruct((NP, D), F32),
        scratch_types=[
            pltpu.VMEM((CHUNK,), jnp.int32),
            pltpu.VMEM((CHUNK, D), F32),
            pltpu.SemaphoreType.DMA,
        ],
    )
    def k(table_hbm, idx_hbm, out_hbm, idx_v, rows_v, sem):
        wid = lax.axis_index("s") * SC_NC + lax.axis_index("c")
        base = wid * CHUNK
        pltpu.sync_copy(idx_hbm.at[pl.ds(base, CHUNK)], idx_v)
        for c in range(CHUNK // SUB):
            pltpu.async_copy(table_hbm.at[idx_v.at[pl.ds(c * SUB, SUB)]],
                             rows_v.at[pl.ds(c * SUB, SUB)], sem).wait()
        pltpu.sync_copy(rows_v, out_hbm.at[pl.ds(base, CHUNK)])

    return k(table, idx_i32)


def kernel(boxes, labels, scores):
    boxes = boxes.astype(F32)
    scores = scores.astype(F32)
    lab_f = labels.astype(F32)

    pad = NP - N
    boxes_p = jnp.concatenate([boxes, jnp.zeros((pad, 4), F32)], axis=0)
    scores_p = jnp.concatenate([scores, jnp.full((pad,), -1.0, F32)], axis=0)
    lab_p = jnp.concatenate([lab_f, jnp.full((pad,), -1.0, F32)], axis=0)

    scol = scores_p.reshape(NP, 1)
    srow = scores_p.reshape(1, NP)
    lcol = lab_p.reshape(NP, 1)

    rank_row, packed = pl.pallas_call(
        _rank_permute_kernel,
        grid=(NR,),
        in_specs=[
            pl.BlockSpec((NP, 1), lambda r: (0, 0)),
            pl.BlockSpec((1, RB), lambda r: (0, r)),
            pl.BlockSpec((RB, 4), lambda r: (r, 0)),
            pl.BlockSpec((RB, 1), lambda r: (r, 0)),
            pl.BlockSpec((RB, 1), lambda r: (r, 0)),
        ],
        out_specs=[
            pl.BlockSpec((1, RB), lambda r: (0, r)),
            pl.BlockSpec((RB, D), lambda r: (r, 0)),
        ],
        out_shape=[
            jax.ShapeDtypeStruct((1, NP), F32),
            jax.ShapeDtypeStruct((NP, D), F32),
        ],
    )(scol, srow, boxes_p, lcol, scol)

    rank_i32 = rank_row.reshape(NP).astype(jnp.int32)
    rank3 = rank_i32.reshape(NW, CHUNK // SUB, SUB)

    sorted_p = _sc_row_scatter_call(packed, rank3)

    sorted_row = sorted_p.T

    keep_row, keep_col = pl.pallas_call(
        _nms_kernel,
        grid=(NB,),
        in_specs=[
            pl.BlockSpec((B, D), lambda b: (b, 0)),
            pl.BlockSpec((D, NP), lambda b: (0, 0)),
            pl.BlockSpec((D, B), lambda b: (0, b)),
        ],
        out_specs=[
            pl.BlockSpec((1, NP), lambda b: (0, 0)),
            pl.BlockSpec((B, 1), lambda b: (b, 0)),
        ],
        out_shape=[
            jax.ShapeDtypeStruct((1, NP), F32),
            jax.ShapeDtypeStruct((NP, 1), F32),
        ],
    )(sorted_p, sorted_row, sorted_row)

    out_sorted = pl.pallas_call(
        _finalize_kernel,
        grid=(1,),
        in_specs=[
            pl.BlockSpec((NP, 1), lambda r: (0, 0)),
            pl.BlockSpec((NP, D), lambda r: (0, 0)),
        ],
        out_specs=pl.BlockSpec((NP, D), lambda r: (0, 0)),
        out_shape=jax.ShapeDtypeStruct((NP, D), F32),
    )(keep_col, sorted_p)

    out_rows = _sc_row_gather_call(out_sorted, rank_i32)

    return out_rows[:N, :5]
